# baseline (speedup 1.0000x reference)
import dataclasses
import functools

import jax
import jax.numpy as jnp
from jax import lax
from jax.experimental import pallas as pl
from jax.experimental.pallas import tpu as pltpu
from jax.experimental.pallas import tpu_sc as plsc

B = 16384
D = 32
NC = 2
NS = 16
NW = NC * NS
BPW = B // NW
L = 16
CH = 128
GRP = BPW // L
EPW = BPW * D


def _fm_body(user_hbm, item_hbm, ue_hbm, ie_hbm, ub_hbm, ib_hbm, gb_hbm,
             out_hbm,
             uidx_v, iidx_v, ueidx_v, ieidx_v, uval_v, ival_v,
             ubias_v, ibias_v, gb_v, out_v,
             sem_u, sem_i, sem_ub, sem_ib):
    c = lax.axis_index("c")
    s = lax.axis_index("s")
    wid = s * NC + c
    base = wid * BPW

    pltpu.sync_copy(user_hbm.at[pl.ds(base, BPW)], uidx_v)
    pltpu.sync_copy(item_hbm.at[pl.ds(base, BPW)], iidx_v)
    pltpu.sync_copy(gb_hbm, gb_v)

    copies = []
    for k in range(BPW // CH):
        sl = pl.ds(k * CH, CH)
        copies.append(pltpu.async_copy(
            ub_hbm.at[uidx_v.at[sl]], ubias_v.at[sl], sem_ub))
        copies.append(pltpu.async_copy(
            ib_hbm.at[iidx_v.at[sl]], ibias_v.at[sl], sem_ib))

    @pl.loop(0, GRP)
    def _(g):
        ub32 = uidx_v[pl.ds(g * L, L)] * D
        ib32 = iidx_v[pl.ds(g * L, L)] * D
        for d in range(D):
            ueidx_v[pl.ds(g * (L * D) + d * L, L)] = ub32 + d
            ieidx_v[pl.ds(g * (L * D) + d * L, L)] = ib32 + d

    for k in range(EPW // CH):
        sl = pl.ds(k * CH, CH)
        copies.append(pltpu.async_copy(
            ue_hbm.at[ueidx_v.at[sl]], uval_v.at[sl], sem_u))
        copies.append(pltpu.async_copy(
            ie_hbm.at[ieidx_v.at[sl]], ival_v.at[sl], sem_i))
    for cp in copies:
        cp.wait()

    gb = gb_v[...]

    @pl.loop(0, GRP)
    def _(g):
        acc = jnp.zeros((L,), jnp.float32)
        for d in range(D):
            off = g * (L * D) + d * L
            acc = acc + uval_v[pl.ds(off, L)] * ival_v[pl.ds(off, L)]
        sl = pl.ds(g * L, L)
        out_v[sl] = acc + ubias_v[sl] + ibias_v[sl] + gb

    pltpu.sync_copy(out_v, out_hbm.at[pl.ds(base, BPW)])


@jax.jit
def _fm(user, item, user_embed, item_embed, user_bias, item_bias, global_bias):
    cp = pltpu.CompilerParams(use_tc_tiling_on_sc=False)
    if "needs_layout_passes" in pltpu.CompilerParams.__dataclass_fields__:
        cp = dataclasses.replace(cp, needs_layout_passes=False)
    run = pl.kernel(
        _fm_body,
        out_type=jax.ShapeDtypeStruct((B,), jnp.float32),
        mesh=plsc.VectorSubcoreMesh(core_axis_name="c", subcore_axis_name="s"),
        compiler_params=cp,
        scratch_types=[
            pltpu.VMEM((BPW,), jnp.int32),
            pltpu.VMEM((BPW,), jnp.int32),
            pltpu.VMEM((EPW,), jnp.int32),
            pltpu.VMEM((EPW,), jnp.int32),
            pltpu.VMEM((EPW,), jnp.float32),
            pltpu.VMEM((EPW,), jnp.float32),
            pltpu.VMEM((BPW,), jnp.float32),
            pltpu.VMEM((BPW,), jnp.float32),
            pltpu.VMEM((L,), jnp.float32),
            pltpu.VMEM((BPW,), jnp.float32),
            pltpu.SemaphoreType.DMA,
            pltpu.SemaphoreType.DMA,
            pltpu.SemaphoreType.DMA,
            pltpu.SemaphoreType.DMA,
        ],
    )
    return run(user, item, user_embed, item_embed, user_bias, item_bias,
               global_bias)


def kernel(user, item, user_embed, item_embed, user_bias, item_bias,
           global_bias):
    return _fm(user.astype(jnp.int32), item.astype(jnp.int32),
               user_embed.reshape(-1), item_embed.reshape(-1),
               user_bias.reshape(-1), item_bias.reshape(-1),
               jnp.broadcast_to(global_bias, (L,)))


# speedup vs baseline: 1.0364x; 1.0364x over previous
import dataclasses
import functools

import jax
import jax.numpy as jnp
from jax import lax
from jax.experimental import pallas as pl
from jax.experimental.pallas import tpu as pltpu
from jax.experimental.pallas import tpu_sc as plsc

B = 16384
D = 32
W = 128
NC = 2
NS = 16
NW = NC * NS
BPW = B // NW
L = 16
CH = 128
NCHUNK = BPW // CH


def _fm_body(user_hbm, item_hbm, ue_hbm, ie_hbm, ub_hbm, ib_hbm, gb_hbm,
             out_hbm,
             uidx_v, iidx_v, urid_v, irid_v, urow_v, irow_v,
             ubias_v, ibias_v, gb_v, out_v,
             sem_u, sem_i, sem_ub, sem_ib):
    c = lax.axis_index("c")
    s = lax.axis_index("s")
    wid = s * NC + c
    base = wid * BPW

    pltpu.sync_copy(user_hbm.at[pl.ds(base, BPW)], uidx_v)
    pltpu.sync_copy(item_hbm.at[pl.ds(base, BPW)], iidx_v)
    pltpu.sync_copy(gb_hbm, gb_v)

    bias_copies = []
    for k in range(NCHUNK):
        sl = pl.ds(k * CH, CH)
        bias_copies.append(pltpu.async_copy(
            ub_hbm.at[uidx_v.at[sl]], ubias_v.at[sl], sem_ub))
        bias_copies.append(pltpu.async_copy(
            ib_hbm.at[iidx_v.at[sl]], ibias_v.at[sl], sem_ib))

    @pl.loop(0, BPW, step=L)
    def _(r0):
        sl = pl.ds(r0, L)
        urid_v[sl] = lax.shift_right_logical(uidx_v[sl], 2)
        irid_v[sl] = lax.shift_right_logical(iidx_v[sl], 2)

    lane = lax.iota(jnp.int32, L)
    low2 = jnp.full((L,), 3, jnp.int32)
    gb = gb_v[...]

    for k in range(NCHUNK):
        sl = pl.ds(k * CH, CH)
        cu = pltpu.async_copy(ue_hbm.at[urid_v.at[sl]], urow_v, sem_u)
        ci = pltpu.async_copy(ie_hbm.at[irid_v.at[sl]], irow_v, sem_i)
        cu.wait()
        ci.wait()

        @pl.loop(0, CH, step=L)
        def _(j0):
            r0 = k * CH + j0
            rows = lane + j0
            ucol0 = (uidx_v[pl.ds(r0, L)] & low2) * D
            icol0 = (iidx_v[pl.ds(r0, L)] & low2) * D
            acc = jnp.zeros((L,), jnp.float32)
            for d in range(D):
                ud = plsc.load_gather(urow_v, [rows, ucol0 + d])
                vd = plsc.load_gather(irow_v, [rows, icol0 + d])
                acc = acc + ud * vd
            out_v[pl.ds(r0, L)] = acc

    for cp in bias_copies:
        cp.wait()

    @pl.loop(0, BPW, step=L)
    def _(r0):
        sl = pl.ds(r0, L)
        out_v[sl] = out_v[sl] + ubias_v[sl] + ibias_v[sl] + gb

    pltpu.sync_copy(out_v, out_hbm.at[pl.ds(base, BPW)])


@jax.jit
def _fm(user, item, user_embed, item_embed, user_bias, item_bias, global_bias):
    cp = pltpu.CompilerParams(use_tc_tiling_on_sc=True)
    if "needs_layout_passes" in pltpu.CompilerParams.__dataclass_fields__:
        cp = dataclasses.replace(cp, needs_layout_passes=False)
    run = pl.kernel(
        _fm_body,
        out_type=jax.ShapeDtypeStruct((B,), jnp.float32),
        mesh=plsc.VectorSubcoreMesh(core_axis_name="c", subcore_axis_name="s"),
        compiler_params=cp,
        scratch_types=[
            pltpu.VMEM((BPW,), jnp.int32),
            pltpu.VMEM((BPW,), jnp.int32),
            pltpu.VMEM((BPW,), jnp.int32),
            pltpu.VMEM((BPW,), jnp.int32),
            pltpu.VMEM((CH, W), jnp.float32),
            pltpu.VMEM((CH, W), jnp.float32),
            pltpu.VMEM((BPW,), jnp.float32),
            pltpu.VMEM((BPW,), jnp.float32),
            pltpu.VMEM((L,), jnp.float32),
            pltpu.VMEM((BPW,), jnp.float32),
            pltpu.SemaphoreType.DMA,
            pltpu.SemaphoreType.DMA,
            pltpu.SemaphoreType.DMA,
            pltpu.SemaphoreType.DMA,
        ],
    )
    return run(user, item, user_embed, item_embed, user_bias, item_bias,
               global_bias)


def kernel(user, item, user_embed, item_embed, user_bias, item_bias,
           global_bias):
    return _fm(user.astype(jnp.int32), item.astype(jnp.int32),
               user_embed.reshape(-1, W), item_embed.reshape(-1, W),
               user_bias.reshape(-1), item_bias.reshape(-1),
               jnp.broadcast_to(global_bias, (L,)))


# speedup vs baseline: 1.0524x; 1.0154x over previous
import dataclasses
import functools

import jax
import jax.numpy as jnp
from jax import lax
from jax.experimental import pallas as pl
from jax.experimental.pallas import tpu as pltpu
from jax.experimental.pallas import tpu_sc as plsc

B = 16384
D = 32
NC = 2
NS = 16
NW = NC * NS
BPW = B // NW
L = 16
CH = 128


def _fm_body(user_hbm, item_hbm, ue_hbm, ie_hbm, ub_hbm, ib_hbm, gb_hbm,
             out_hbm,
             uidx_v, iidx_v, urows_v, irows_v, ubias_v, ibias_v, gb_v, out_v,
             sem_u, sem_i, sem_ub, sem_ib):
    c = lax.axis_index("c")
    s = lax.axis_index("s")
    wid = s * NC + c
    base = wid * BPW

    pltpu.sync_copy(user_hbm.at[pl.ds(base, BPW)], uidx_v)
    pltpu.sync_copy(item_hbm.at[pl.ds(base, BPW)], iidx_v)
    pltpu.sync_copy(gb_hbm, gb_v)

    copies = []
    for k in range(BPW // CH):
        sl = pl.ds(k * CH, CH)
        copies.append(pltpu.async_copy(
            ue_hbm.at[uidx_v.at[sl]], urows_v.at[sl], sem_u))
        copies.append(pltpu.async_copy(
            ie_hbm.at[iidx_v.at[sl]], irows_v.at[sl], sem_i))
        copies.append(pltpu.async_copy(
            ub_hbm.at[uidx_v.at[sl]], ubias_v.at[sl], sem_ub))
        copies.append(pltpu.async_copy(
            ib_hbm.at[iidx_v.at[sl]], ibias_v.at[sl], sem_ib))
    for cp in copies:
        cp.wait()

    lane = lax.iota(jnp.int32, L)
    gb = gb_v[...]

    @pl.loop(0, BPW, step=L)
    def _(r0):
        rows = lane + r0
        acc = jnp.zeros((L,), jnp.float32)
        for d in range(D):
            col = jnp.full((L,), d, jnp.int32)
            ud = plsc.load_gather(urows_v, [rows, col])
            vd = plsc.load_gather(irows_v, [rows, col])
            acc = acc + ud * vd
        sl = pl.ds(r0, L)
        out_v[sl] = acc + ubias_v[sl] + ibias_v[sl] + gb

    pltpu.sync_copy(out_v, out_hbm.at[pl.ds(base, BPW)])


@jax.jit
def _fm(user, item, user_embed, item_embed, user_bias, item_bias, global_bias):
    cp = pltpu.CompilerParams(use_tc_tiling_on_sc=False)
    if "needs_layout_passes" in pltpu.CompilerParams.__dataclass_fields__:
        cp = dataclasses.replace(cp, needs_layout_passes=False)
    run = pl.kernel(
        _fm_body,
        out_type=jax.ShapeDtypeStruct((B,), jnp.float32),
        mesh=plsc.VectorSubcoreMesh(core_axis_name="c", subcore_axis_name="s"),
        compiler_params=cp,
        scratch_types=[
            pltpu.VMEM((BPW,), jnp.int32),
            pltpu.VMEM((BPW,), jnp.int32),
            pltpu.VMEM((BPW, D), jnp.float32),
            pltpu.VMEM((BPW, D), jnp.float32),
            pltpu.VMEM((BPW,), jnp.float32),
            pltpu.VMEM((BPW,), jnp.float32),
            pltpu.VMEM((L,), jnp.float32),
            pltpu.VMEM((BPW,), jnp.float32),
            pltpu.SemaphoreType.DMA,
            pltpu.SemaphoreType.DMA,
            pltpu.SemaphoreType.DMA,
            pltpu.SemaphoreType.DMA,
        ],
    )
    return run(user, item, user_embed, item_embed, user_bias, item_bias,
               global_bias)


def kernel(user, item, user_embed, item_embed, user_bias, item_bias,
           global_bias):
    return _fm(user.astype(jnp.int32), item.astype(jnp.int32),
               user_embed, item_embed,
               user_bias.reshape(-1), item_bias.reshape(-1),
               jnp.broadcast_to(global_bias, (L,)))
